# baseline (speedup 1.0000x reference)
import functools

import jax
import jax.numpy as jnp
from jax import lax
from jax.experimental import pallas as pl
from jax.experimental.pallas import tpu as pltpu
from jax.experimental.pallas import tpu_sc as plsc

NC = 2
NS = 16
LANES = 16
CHUNK = 256


def _mm_body(x_ref, w_ref, o_ref):
    o_ref[...] = jnp.dot(x_ref[...], w_ref[...], preferred_element_type=jnp.float32)


def _combine_body(p_ref, b_ref, o_ref):
    o_ref[...] = p_ref[0] + p_ref[1] + b_ref[...]


def _make_sc_agg(n_nodes, chunks_per_tile, d):
    acc_rows = n_nodes
    rows_per_tile = (n_nodes // NS) & ~7
    rows_tail = n_nodes - NS * rows_per_tile
    mesh = plsc.VectorSubcoreMesh(core_axis_name="c", subcore_axis_name="s")

    @functools.partial(
        pl.kernel,
        out_type=jax.ShapeDtypeStruct((NC, n_nodes, d), jnp.float32),
        mesh=mesh,
        scratch_types=[
            pltpu.VMEM((CHUNK,), jnp.int32),
            pltpu.VMEM((CHUNK,), jnp.int32),
            pltpu.VMEM((CHUNK, d), jnp.float32),
            pltpu.VMEM_SHARED((acc_rows, d), jnp.float32),
            pltpu.SemaphoreType.DMA,
        ],
    )
    def sc_agg(y_hbm, src_hbm, dst_hbm, out_hbm,
               src_v, dst_v, buf_v, acc_sh, sem):
        c = lax.axis_index("c")
        s = lax.axis_index("s")
        w = c * NS + s

        def zero_row(i, carry):
            for j in range(d // LANES):
                buf_v[i, pl.ds(j * LANES, LANES)] = jnp.zeros((LANES,), jnp.float32)
            return carry
        lax.fori_loop(0, CHUNK, zero_row, 0)

        row_base = s * rows_per_tile
        n_full = rows_per_tile // CHUNK
        for k in range(n_full):
            pltpu.sync_copy(buf_v, acc_sh.at[pl.ds(row_base + k * CHUNK, CHUNK)])
        tail = rows_per_tile - n_full * CHUNK
        if tail:
            pltpu.sync_copy(buf_v.at[pl.ds(0, tail)],
                            acc_sh.at[pl.ds(row_base + n_full * CHUNK, tail)])
        if rows_tail:
            @pl.when(s == NS - 1)
            def _zero_last_rows():
                pltpu.sync_copy(buf_v.at[pl.ds(0, rows_tail)],
                                acc_sh.at[pl.ds(NS * rows_per_tile, rows_tail)])
        plsc.subcore_barrier()

        base0 = w * chunks_per_tile * CHUNK

        def chunk_one(i, carry):
            eb = base0 + i * CHUNK
            pltpu.sync_copy(src_hbm.at[pl.ds(eb, CHUNK)], src_v)
            pltpu.sync_copy(dst_hbm.at[pl.ds(eb, CHUNK)], dst_v)
            pltpu.async_copy(y_hbm.at[src_v], buf_v, sem).wait()
            pltpu.sync_copy(buf_v, acc_sh.at[dst_v], add=True)
            return carry
        lax.fori_loop(0, chunks_per_tile, chunk_one, 0)

        plsc.subcore_barrier()
        pltpu.sync_copy(acc_sh.at[pl.ds(row_base, rows_per_tile)],
                        out_hbm.at[c, pl.ds(row_base, rows_per_tile)])
        if rows_tail:
            @pl.when(s == NS - 1)
            def _copy_last_rows():
                pltpu.sync_copy(acc_sh.at[pl.ds(NS * rows_per_tile, rows_tail)],
                                out_hbm.at[c, pl.ds(NS * rows_per_tile, rows_tail)])

    return sc_agg


def kernel(x, edge_index, W, b):
    n_nodes, d_in = x.shape
    d_out = W.shape[1]
    n_edges = edge_index.shape[1]

    src = edge_index[1].astype(jnp.int32)
    dst = edge_index[0].astype(jnp.int32)

    nw = NC * NS
    cpt = -(-n_edges // (nw * CHUNK))
    n_pad = nw * cpt * CHUNK - n_edges
    src_p = jnp.concatenate([src, jnp.full((n_pad,), n_nodes, jnp.int32)])
    dst_p = jnp.concatenate([dst, jnp.arange(n_pad, dtype=jnp.int32) % n_nodes])

    row_blk = 1024
    mm_rows = -(-(n_nodes + 1) // row_blk) * row_blk
    x_p = jnp.concatenate(
        [x, jnp.zeros((mm_rows - n_nodes, d_in), jnp.float32)])
    y = pl.pallas_call(
        _mm_body,
        grid=(mm_rows // row_blk,),
        in_specs=[pl.BlockSpec((row_blk, d_in), lambda i: (i, 0)),
                  pl.BlockSpec((d_in, d_out), lambda i: (0, 0))],
        out_specs=pl.BlockSpec((row_blk, d_out), lambda i: (i, 0)),
        out_shape=jax.ShapeDtypeStruct((mm_rows, d_out), jnp.float32),
    )(x_p, W)

    partials = _make_sc_agg(n_nodes, cpt, d_out)(y, src_p, dst_p)

    cb_blk = 1000
    out = pl.pallas_call(
        _combine_body,
        grid=(n_nodes // cb_blk,),
        in_specs=[pl.BlockSpec((NC, cb_blk, d_out), lambda i: (0, i, 0)),
                  pl.BlockSpec((1, d_out), lambda i: (0, 0))],
        out_specs=pl.BlockSpec((cb_blk, d_out), lambda i: (i, 0)),
        out_shape=jax.ShapeDtypeStruct((n_nodes, d_out), jnp.float32),
    )(partials, b.reshape(1, d_out))
    return out


# speedup vs baseline: 2.2708x; 2.2708x over previous
import functools

import jax
import jax.numpy as jnp
from jax import lax
from jax.experimental import pallas as pl
from jax.experimental.pallas import tpu as pltpu
from jax.experimental.pallas import tpu_sc as plsc

NC = 2
NS = 16
LANES = 16


def _mm_body(x_ref, w_ref, o_ref):
    o_ref[...] = jnp.dot(x_ref[...], w_ref[...], preferred_element_type=jnp.float32)


def _combine_body(p_ref, b_ref, o_ref):
    o_ref[...] = p_ref[0] + p_ref[1] + b_ref[...]


def _make_sc_agg(n_nodes, n_edges, d):
    nw = NC * NS
    edges_per_tile = n_edges // nw
    chunk = 128
    full_chunks = edges_per_tile // chunk
    rem = edges_per_tile - full_chunks * chunk
    rows_per_tile = (n_nodes // NS) & ~7
    rows_tail = n_nodes - NS * rows_per_tile
    mesh = plsc.VectorSubcoreMesh(core_axis_name="c", subcore_axis_name="s")

    @functools.partial(
        pl.kernel,
        out_type=jax.ShapeDtypeStruct((NC, n_nodes, d), jnp.float32),
        mesh=mesh,
        scratch_types=[
            pltpu.VMEM((chunk,), jnp.int32),
            pltpu.VMEM((chunk,), jnp.int32),
            pltpu.VMEM((chunk, d), jnp.float32),
            pltpu.VMEM((rem,), jnp.int32),
            pltpu.VMEM((rem,), jnp.int32),
            pltpu.VMEM((rem, d), jnp.float32),
            pltpu.VMEM_SHARED((n_nodes, d), jnp.float32),
            pltpu.SemaphoreType.DMA,
        ],
    )
    def sc_agg(y_hbm, src_hbm, dst_hbm, out_hbm,
               src_v, dst_v, rows_v, src_r, dst_r, rows_r, acc_sh, sem):
        c = lax.axis_index("c")
        s = lax.axis_index("s")

        def zero_row(i, carry):
            for j in range(d // LANES):
                rows_v[i, pl.ds(j * LANES, LANES)] = jnp.zeros((LANES,), jnp.float32)
            return carry
        lax.fori_loop(0, chunk, zero_row, 0)

        row_base = s * rows_per_tile
        n_full = rows_per_tile // chunk
        for k in range(n_full):
            pltpu.sync_copy(rows_v, acc_sh.at[pl.ds(row_base + k * chunk, chunk)])
        tail = rows_per_tile - n_full * chunk
        if tail:
            pltpu.sync_copy(rows_v.at[pl.ds(0, tail)],
                            acc_sh.at[pl.ds(row_base + n_full * chunk, tail)])
        if rows_tail:
            @pl.when(s == NS - 1)
            def _zero_last_rows():
                pltpu.sync_copy(rows_v.at[pl.ds(0, rows_tail)],
                                acc_sh.at[pl.ds(NS * rows_per_tile, rows_tail)])
        plsc.subcore_barrier()

        base0 = (c * NS + s) * edges_per_tile

        def chunk_body(i, carry):
            eb = base0 + i * chunk
            pltpu.sync_copy(src_hbm.at[pl.ds(eb, chunk)], src_v)
            pltpu.sync_copy(dst_hbm.at[pl.ds(eb, chunk)], dst_v)
            pltpu.async_copy(y_hbm.at[src_v], rows_v, sem).wait()
            pltpu.sync_copy(rows_v, acc_sh.at[dst_v], add=True)
            return carry
        lax.fori_loop(0, full_chunks, chunk_body, 0)

        if rem:
            eb = base0 + full_chunks * chunk
            pltpu.sync_copy(src_hbm.at[pl.ds(eb, rem)], src_r)
            pltpu.sync_copy(dst_hbm.at[pl.ds(eb, rem)], dst_r)
            pltpu.async_copy(y_hbm.at[src_r], rows_r, sem).wait()
            pltpu.sync_copy(rows_r, acc_sh.at[dst_r], add=True)

        plsc.subcore_barrier()
        pltpu.sync_copy(acc_sh.at[pl.ds(row_base, rows_per_tile)],
                        out_hbm.at[c, pl.ds(row_base, rows_per_tile)])
        if rows_tail:
            @pl.when(s == NS - 1)
            def _copy_last_rows():
                pltpu.sync_copy(acc_sh.at[pl.ds(NS * rows_per_tile, rows_tail)],
                                out_hbm.at[c, pl.ds(NS * rows_per_tile, rows_tail)])

    return sc_agg


def kernel(x, edge_index, W, b):
    n_nodes, d_in = x.shape
    d_out = W.shape[1]
    n_edges = edge_index.shape[1]

    src = edge_index[1].astype(jnp.int32)
    dst = edge_index[0].astype(jnp.int32)

    row_blk = 1000
    y = pl.pallas_call(
        _mm_body,
        grid=(n_nodes // row_blk,),
        in_specs=[pl.BlockSpec((row_blk, d_in), lambda i: (i, 0)),
                  pl.BlockSpec((d_in, d_out), lambda i: (0, 0))],
        out_specs=pl.BlockSpec((row_blk, d_out), lambda i: (i, 0)),
        out_shape=jax.ShapeDtypeStruct((n_nodes, d_out), jnp.float32),
    )(x, W)

    partials = _make_sc_agg(n_nodes, n_edges, d_out)(y, src, dst)

    out = pl.pallas_call(
        _combine_body,
        grid=(n_nodes // row_blk,),
        in_specs=[pl.BlockSpec((NC, row_blk, d_out), lambda i: (0, i, 0)),
                  pl.BlockSpec((1, d_out), lambda i: (0, 0))],
        out_specs=pl.BlockSpec((row_blk, d_out), lambda i: (i, 0)),
        out_shape=jax.ShapeDtypeStruct((n_nodes, d_out), jnp.float32),
    )(partials, b.reshape(1, d_out))
    return out
